# baseline (speedup 1.0000x reference)
import functools

import jax
import jax.numpy as jnp
from jax import lax
from jax.experimental import pallas as pl
from jax.experimental.pallas import tpu as pltpu
from jax.experimental.pallas import tpu_sc as plsc

_VOCAB = 1000
_DPAD = 1024
_NC = 2
_NS = 16
_NW = _NC * _NS
_CHUNK = 40
_NBUF = 2
_TAIL = _VOCAB - 7 * 128


def _sc_gather(idx_flat, table_p):
    n = idx_flat.shape[0]
    b_per_w = n // _NW
    n_chunks = b_per_w // _CHUNK
    mesh = plsc.VectorSubcoreMesh(core_axis_name="c", subcore_axis_name="s")

    @functools.partial(
        pl.kernel,
        mesh=mesh,
        out_type=jax.ShapeDtypeStruct((n, _VOCAB), jnp.float32),
        scratch_types=[
            pltpu.VMEM((b_per_w,), jnp.int32),
            pltpu.VMEM((_NBUF, _CHUNK, _DPAD), jnp.float32),
            pltpu.VMEM((_NBUF, _CHUNK, _TAIL), jnp.float32),
            pltpu.SemaphoreType.DMA,
            pltpu.SemaphoreType.DMA,
            pltpu.SemaphoreType.DMA,
            pltpu.SemaphoreType.DMA,
        ],
    )
    def k(idx_hbm, table_hbm, out_hbm, idx_v, gbuf, tbuf, gs0, gs1,
          ws0, ws1):
        gsems = [gs0, gs1]
        wsems = [ws0, ws1]
        cid = lax.axis_index("c")
        sid = lax.axis_index("s")
        wid = sid * _NC + cid
        base = wid * b_per_w

        pltpu.sync_copy(idx_hbm.at[pl.ds(base, b_per_w)], idx_v)

        def start_g(c, b):
            pltpu.async_copy(
                table_hbm.at[idx_v.at[pl.ds(c * _CHUNK, _CHUNK)]],
                gbuf.at[b],
                gsems[b],
            )

        def wait_g(c, b):
            pltpu.make_async_copy(
                table_hbm.at[idx_v.at[pl.ds(c * _CHUNK, _CHUNK)]],
                gbuf.at[b],
                gsems[b],
            ).wait()

        tail_offs = [0, 16, 32, 48, 64, 80, _TAIL - 16]

        def write(c, b):
            row = base + c * _CHUNK
            for r in range(_CHUNK):
                for o in tail_offs:
                    tbuf[b, r, pl.ds(o, 16)] = gbuf[b, r, pl.ds(896 + o, 16)]
            for t in range(7):
                pltpu.async_copy(
                    gbuf.at[b, :, pl.ds(t * 128, 128)],
                    out_hbm.at[pl.ds(row, _CHUNK), pl.ds(t * 128, 128)],
                    wsems[b],
                )
            pltpu.async_copy(
                tbuf.at[b],
                out_hbm.at[pl.ds(row, _CHUNK), pl.ds(896, _TAIL)],
                wsems[b],
            )

        def wait_w(c, b):
            row = base + c * _CHUNK
            for t in range(7):
                pltpu.make_async_copy(
                    gbuf.at[b, :, pl.ds(t * 128, 128)],
                    out_hbm.at[pl.ds(row, _CHUNK), pl.ds(t * 128, 128)],
                    wsems[b],
                ).wait()
            pltpu.make_async_copy(
                tbuf.at[b],
                out_hbm.at[pl.ds(row, _CHUNK), pl.ds(896, _TAIL)],
                wsems[b],
            ).wait()

        for b in range(_NBUF):
            start_g(b, b)

        def outer(g, carry):
            for b in range(_NBUF):
                c = g * _NBUF + b
                wait_g(c, b)
                write(c, b)
                wait_w(c, b)
                start_g(c + _NBUF, b)
            return carry

        lax.fori_loop(0, (n_chunks - _NBUF) // _NBUF, outer, 0)

        for b in range(_NBUF):
            c = n_chunks - _NBUF + b
            wait_g(c, b)
            write(c, b)
            wait_w(c, b)

    return k(idx_flat, table_p)


_PIECES = 4


def kernel(idx, table):
    b, s = idx.shape
    n = b * s
    idx_flat = idx.reshape(n).astype(jnp.int32)
    table_p = jnp.pad(table.astype(jnp.float32), ((0, 0), (0, _DPAD - _VOCAB)))
    np_ = n // _PIECES
    outs = [
        _sc_gather(lax.slice(idx_flat, (p * np_,), ((p + 1) * np_,)), table_p)
        for p in range(_PIECES)
    ]
    return jnp.concatenate(outs, axis=0)


# speedup vs baseline: 1.4205x; 1.4205x over previous
import functools

import jax
import jax.numpy as jnp
from jax import lax
from jax.experimental import pallas as pl
from jax.experimental.pallas import tpu as pltpu
from jax.experimental.pallas import tpu_sc as plsc

_VOCAB = 1000
_DPAD = 1024
_NC = 2
_NS = 16
_NW = _NC * _NS
_CHUNK = 48
_NBUF = 2
_TAIL = _VOCAB - 7 * 128
_REM = 16


def _sc_gather(idx_flat, table_p):
    n = idx_flat.shape[0]
    b_per_w = n // _NW
    n_chunks = (b_per_w - _REM) // _CHUNK
    mesh = plsc.VectorSubcoreMesh(core_axis_name="c", subcore_axis_name="s")

    @functools.partial(
        pl.kernel,
        mesh=mesh,
        out_type=jax.ShapeDtypeStruct((n, _VOCAB), jnp.float32),
        scratch_types=[
            pltpu.VMEM((b_per_w,), jnp.int32),
            pltpu.VMEM((_NBUF, _CHUNK, _DPAD), jnp.float32),
            pltpu.VMEM((_NBUF, _CHUNK, _TAIL), jnp.float32),
            pltpu.SemaphoreType.DMA,
            pltpu.SemaphoreType.DMA,
            pltpu.SemaphoreType.DMA,
            pltpu.SemaphoreType.DMA,
        ],
    )
    def k(idx_hbm, table_hbm, out_hbm, idx_v, gbuf, tbuf, gs0, gs1,
          ws0, ws1):
        gsems = [gs0, gs1]
        wsems = [ws0, ws1]
        cid = lax.axis_index("c")
        sid = lax.axis_index("s")
        wid = sid * _NC + cid
        base = wid * b_per_w

        pltpu.sync_copy(idx_hbm.at[pl.ds(base, b_per_w)], idx_v)

        def start_g(c, b, nrows=_CHUNK):
            pltpu.async_copy(
                table_hbm.at[idx_v.at[pl.ds(c * _CHUNK, nrows)]],
                gbuf.at[b, pl.ds(0, nrows)],
                gsems[b],
            )

        def wait_g(c, b, nrows=_CHUNK):
            pltpu.make_async_copy(
                table_hbm.at[idx_v.at[pl.ds(c * _CHUNK, nrows)]],
                gbuf.at[b, pl.ds(0, nrows)],
                gsems[b],
            ).wait()

        tail_offs = [0, 16, 32, 48, 64, 80, _TAIL - 16]

        def write(c, b, nrows=_CHUNK):
            row = base + c * _CHUNK
            for t in range(7):
                pltpu.async_copy(
                    gbuf.at[b, pl.ds(0, nrows), pl.ds(t * 128, 128)],
                    out_hbm.at[pl.ds(row, nrows), pl.ds(t * 128, 128)],
                    wsems[b],
                )
            for r in range(nrows):
                for o in tail_offs:
                    tbuf[b, r, pl.ds(o, 16)] = gbuf[b, r, pl.ds(896 + o, 16)]
            pltpu.async_copy(
                tbuf.at[b, pl.ds(0, nrows)],
                out_hbm.at[pl.ds(row, nrows), pl.ds(896, _TAIL)],
                wsems[b],
            )

        def wait_w(c, b, nrows=_CHUNK):
            row = base + c * _CHUNK
            for t in range(7):
                pltpu.make_async_copy(
                    gbuf.at[b, pl.ds(0, nrows), pl.ds(t * 128, 128)],
                    out_hbm.at[pl.ds(row, nrows), pl.ds(t * 128, 128)],
                    wsems[b],
                ).wait()
            pltpu.make_async_copy(
                tbuf.at[b, pl.ds(0, nrows)],
                out_hbm.at[pl.ds(row, nrows), pl.ds(896, _TAIL)],
                wsems[b],
            ).wait()

        for b in range(_NBUF):
            start_g(b, b)

        def outer(g, carry):
            for b in range(_NBUF):
                c = g * _NBUF + b
                wait_g(c, b)
                write(c, b)
                wait_w(c, b)
                start_g(c + _NBUF, b)
            return carry

        n_main = ((n_chunks - _NBUF) // _NBUF) * _NBUF
        lax.fori_loop(0, n_main // _NBUF, outer, 0)

        tail_work = [(c, _CHUNK) for c in range(n_main, n_chunks)]
        tail_work.append((n_chunks, _REM))
        for i, (c, nr) in enumerate(tail_work):
            b = c % _NBUF
            wait_g(c, b, nrows=nr)
            write(c, b, nrows=nr)
            wait_w(c, b, nrows=nr)
            if i + _NBUF < len(tail_work):
                cn, nn = tail_work[i + _NBUF]
                start_g(cn, cn % _NBUF, nrows=nn)

    return k(idx_flat, table_p)


def kernel(idx, table):
    b, s = idx.shape
    idx_flat = idx.reshape(b * s).astype(jnp.int32)
    table_p = jnp.pad(table.astype(jnp.float32), ((0, 0), (0, _DPAD - _VOCAB)))
    return _sc_gather(idx_flat, table_p)


# speedup vs baseline: 1.4270x; 1.0046x over previous
import functools

import jax
import jax.numpy as jnp
from jax import lax
from jax.experimental import pallas as pl
from jax.experimental.pallas import tpu as pltpu
from jax.experimental.pallas import tpu_sc as plsc

_VOCAB = 1000
_DPAD = 1024
_NC = 2
_NS = 16
_NW = _NC * _NS
_CHUNK = 40
_NBUF = 2
_TAIL = _VOCAB - 7 * 128
_REM = 0


def _sc_gather(idx_flat, table_p):
    n = idx_flat.shape[0]
    b_per_w = n // _NW
    n_chunks = (b_per_w - _REM) // _CHUNK
    mesh = plsc.VectorSubcoreMesh(core_axis_name="c", subcore_axis_name="s")

    @functools.partial(
        pl.kernel,
        mesh=mesh,
        out_type=jax.ShapeDtypeStruct((n, _VOCAB), jnp.float32),
        scratch_types=[
            pltpu.VMEM((b_per_w,), jnp.int32),
            pltpu.VMEM((_NBUF, _CHUNK, _DPAD), jnp.float32),
            pltpu.VMEM((_NBUF, _CHUNK, _TAIL), jnp.float32),
            pltpu.SemaphoreType.DMA,
            pltpu.SemaphoreType.DMA,
            pltpu.SemaphoreType.DMA,
            pltpu.SemaphoreType.DMA,
        ],
    )
    def k(idx_hbm, table_hbm, out_hbm, idx_v, gbuf, tbuf, gs0, gs1,
          ws0, ws1):
        gsems = [gs0, gs1]
        wsems = [ws0, ws1]
        cid = lax.axis_index("c")
        sid = lax.axis_index("s")
        wid = sid * _NC + cid
        base = wid * b_per_w

        pltpu.sync_copy(idx_hbm.at[pl.ds(base, b_per_w)], idx_v)

        def start_g(c, b, nrows=_CHUNK):
            pltpu.async_copy(
                table_hbm.at[idx_v.at[pl.ds(c * _CHUNK, nrows)]],
                gbuf.at[b, pl.ds(0, nrows)],
                gsems[b],
            )

        def wait_g(c, b, nrows=_CHUNK):
            pltpu.make_async_copy(
                table_hbm.at[idx_v.at[pl.ds(c * _CHUNK, nrows)]],
                gbuf.at[b, pl.ds(0, nrows)],
                gsems[b],
            ).wait()

        tail_offs = [0, 16, 32, 48, 64, 80, _TAIL - 16]

        def write(c, b, nrows=_CHUNK):
            row = base + c * _CHUNK
            for t in range(7):
                pltpu.async_copy(
                    gbuf.at[b, pl.ds(0, nrows), pl.ds(t * 128, 128)],
                    out_hbm.at[pl.ds(row, nrows), pl.ds(t * 128, 128)],
                    wsems[b],
                )
            for r in range(nrows):
                for o in tail_offs:
                    tbuf[b, r, pl.ds(o, 16)] = gbuf[b, r, pl.ds(896 + o, 16)]
            pltpu.async_copy(
                tbuf.at[b, pl.ds(0, nrows)],
                out_hbm.at[pl.ds(row, nrows), pl.ds(896, _TAIL)],
                wsems[b],
            )

        def wait_w(c, b, nrows=_CHUNK):
            row = base + c * _CHUNK
            for t in range(7):
                pltpu.make_async_copy(
                    gbuf.at[b, pl.ds(0, nrows), pl.ds(t * 128, 128)],
                    out_hbm.at[pl.ds(row, nrows), pl.ds(t * 128, 128)],
                    wsems[b],
                ).wait()
            pltpu.make_async_copy(
                tbuf.at[b, pl.ds(0, nrows)],
                out_hbm.at[pl.ds(row, nrows), pl.ds(896, _TAIL)],
                wsems[b],
            ).wait()

        for b in range(_NBUF):
            start_g(b, b)

        def outer(g, carry):
            for b in range(_NBUF):
                c = g * _NBUF + b
                wait_g(c, b)
                write(c, b)
                wait_w(c, b)
                start_g(c + _NBUF, b)
            return carry

        n_main = ((n_chunks - _NBUF) // _NBUF) * _NBUF
        lax.fori_loop(0, n_main // _NBUF, outer, 0)

        tail_work = [(c, _CHUNK) for c in range(n_main, n_chunks)]
        if _REM:
            tail_work.append((n_chunks, _REM))
        for i, (c, nr) in enumerate(tail_work):
            b = c % _NBUF
            wait_g(c, b, nrows=nr)
            write(c, b, nrows=nr)
            wait_w(c, b, nrows=nr)
            if i + _NBUF < len(tail_work):
                cn, nn = tail_work[i + _NBUF]
                start_g(cn, cn % _NBUF, nrows=nn)

    return k(idx_flat, table_p)


def kernel(idx, table):
    b, s = idx.shape
    idx_flat = idx.reshape(b * s).astype(jnp.int32)
    table_p = jnp.pad(table.astype(jnp.float32), ((0, 0), (0, _DPAD - _VOCAB)))
    return _sc_gather(idx_flat, table_p)
